# baseline (speedup 1.0000x reference)
import functools

import jax
import jax.numpy as jnp
from jax import lax
from jax.experimental import pallas as pl
from jax.experimental.pallas import tpu as pltpu
from jax.experimental.pallas import tpu_sc as plsc

VOCAB = 1000000
EMB_D = 32
BATCH = 4096
HIST = 200

_info = plsc.get_sparse_core_info()
NC = _info.num_cores
NS = _info.num_subcores
LANES = _info.num_lanes
NW = NC * NS
BPW = BATCH // NW
SC_ROWS = NS * BPW
NCHUNK = BPW * HIST // BPW


def _make_kernel():
    mesh = plsc.VectorSubcoreMesh(core_axis_name="c", subcore_axis_name="s")

    @functools.partial(
        pl.kernel,
        mesh=mesh,
        out_type=jax.ShapeDtypeStruct((BATCH, EMB_D), jnp.float32),
        compiler_params=pltpu.CompilerParams(use_tc_tiling_on_sc=False),
        scratch_types=[
            pltpu.VMEM((NCHUNK, BPW), jnp.int32),
            pltpu.VMEM((NCHUNK, BPW), jnp.int32),
            pltpu.VMEM((BPW, EMB_D), jnp.float32),
            pltpu.VMEM((BPW, EMB_D), jnp.float32),
            pltpu.VMEM_SHARED((SC_ROWS, EMB_D), jnp.float32),
            pltpu.SemaphoreType.DMA,
            pltpu.SemaphoreType.DMA,
        ],
    )
    def k(xs_hbm, dpat_hbm, table_hbm, out_hbm,
          idx_v, dst_v, buf_a, buf_b, acc, sem_a, sem_b):
        c = lax.axis_index("c")
        s = lax.axis_index("s")
        w = c * NS + s

        zero = jnp.zeros((LANES,), jnp.float32)

        def zbody(j, carry):
            buf_a[j, pl.ds(0, LANES)] = zero
            buf_a[j, pl.ds(LANES, LANES)] = zero
            return carry

        lax.fori_loop(0, BPW, zbody, 0)
        pltpu.sync_copy(buf_a, acc.at[pl.ds(s * BPW, BPW)])

        pltpu.sync_copy(xs_hbm.at[w], idx_v)
        pltpu.sync_copy(dpat_hbm.at[s], dst_v)

        def g_start(t, buf, sem):
            pltpu.async_copy(table_hbm.at[idx_v.at[t]], buf, sem)

        def g_wait(t, buf, sem):
            pltpu.make_async_copy(table_hbm.at[idx_v.at[t]], buf, sem).wait()

        def scat(t, buf):
            pltpu.sync_copy(buf, acc.at[dst_v.at[t]], add=True)

        g_start(0, buf_a, sem_a)
        g_start(1, buf_b, sem_b)

        def body(kk, carry):
            ta = 2 * kk
            g_wait(ta, buf_a, sem_a)
            scat(ta, buf_a)
            g_start(ta + 2, buf_a, sem_a)
            g_wait(ta + 1, buf_b, sem_b)
            scat(ta + 1, buf_b)
            g_start(ta + 3, buf_b, sem_b)
            return carry

        lax.fori_loop(0, NCHUNK // 2 - 1, body, 0)

        g_wait(NCHUNK - 2, buf_a, sem_a)
        scat(NCHUNK - 2, buf_a)
        g_wait(NCHUNK - 1, buf_b, sem_b)
        scat(NCHUNK - 1, buf_b)

        pltpu.sync_copy(acc.at[pl.ds(s * BPW, BPW)], buf_a)
        inv = jnp.float32(1.0 / HIST)

        def sbody(j, carry):
            buf_a[j, pl.ds(0, LANES)] = buf_a[j, pl.ds(0, LANES)] * inv
            buf_a[j, pl.ds(LANES, LANES)] = buf_a[j, pl.ds(LANES, LANES)] * inv
            return carry

        lax.fori_loop(0, BPW, sbody, 0)
        pltpu.sync_copy(buf_a, out_hbm.at[pl.ds(w * BPW, BPW)])

    return k


_sc_kernel = _make_kernel()


def kernel(xs, table):
    xs_r = xs.astype(jnp.int32).reshape(NW, NCHUNK, BPW)
    p = jnp.arange(BPW * HIST, dtype=jnp.int32) // HIST
    dpat = (jnp.arange(NS, dtype=jnp.int32)[:, None] * BPW
            + p[None, :]).reshape(NS, NCHUNK, BPW)
    return _sc_kernel(xs_r, dpat, table)


# speedup vs baseline: 1.1280x; 1.1280x over previous
import functools

import jax
import jax.numpy as jnp
from jax import lax
from jax.experimental import pallas as pl
from jax.experimental.pallas import tpu as pltpu
from jax.experimental.pallas import tpu_sc as plsc

VOCAB = 1000000
EMB_D = 32
BATCH = 4096
HIST = 200

_info = plsc.get_sparse_core_info()
NC = _info.num_cores
NS = _info.num_subcores
LANES = _info.num_lanes
NW = NC * NS
BPW = BATCH // NW
NBUF = 4
SPLIT = 104


def _make_kernel():
    mesh = plsc.VectorSubcoreMesh(core_axis_name="c", subcore_axis_name="s")

    @functools.partial(
        pl.kernel,
        mesh=mesh,
        out_type=jax.ShapeDtypeStruct((BATCH, EMB_D), jnp.float32),
        compiler_params=pltpu.CompilerParams(use_tc_tiling_on_sc=False),
        scratch_types=[
            pltpu.VMEM((BPW * HIST,), jnp.int32),
            pltpu.VMEM((NBUF, HIST, EMB_D), jnp.float32),
            pltpu.VMEM((BPW, EMB_D), jnp.float32),
            pltpu.SemaphoreType.DMA,
            pltpu.SemaphoreType.DMA,
            pltpu.SemaphoreType.DMA,
            pltpu.SemaphoreType.DMA,
        ],
    )
    def k(xs_hbm, table_hbm, out_hbm, idx_v, bufs, obuf, s0, s1, s2, s3):
        c = lax.axis_index("c")
        s = lax.axis_index("s")
        w = c * NS + s
        sems = (s0, s1, s2, s3)

        pltpu.sync_copy(xs_hbm.at[w], idx_v)

        def descs(r, p):
            src_a = table_hbm.at[idx_v.at[pl.ds(r * HIST, SPLIT)]]
            src_b = table_hbm.at[idx_v.at[pl.ds(r * HIST + SPLIT, HIST - SPLIT)]]
            dst_a = bufs.at[p, pl.ds(0, SPLIT)]
            dst_b = bufs.at[p, pl.ds(SPLIT, HIST - SPLIT)]
            return (src_a, dst_a), (src_b, dst_b)

        def g_start(r, p):
            for src, dst in descs(r, p):
                pltpu.async_copy(src, dst, sems[p])

        def g_wait(r, p):
            for src, dst in descs(r, p):
                pltpu.make_async_copy(src, dst, sems[p]).wait()

        zero = jnp.zeros((LANES,), jnp.float32)
        inv = jnp.float32(1.0 / HIST)

        def reduce_row(r, p):
            def rbody(m, carry):
                ae, ao, be, bo = carry
                base = m * 8
                for u in range(0, 8, 2):
                    ae = ae + bufs[p, base + u, pl.ds(0, LANES)]
                    be = be + bufs[p, base + u, pl.ds(LANES, LANES)]
                    ao = ao + bufs[p, base + u + 1, pl.ds(0, LANES)]
                    bo = bo + bufs[p, base + u + 1, pl.ds(LANES, LANES)]
                return ae, ao, be, bo

            ae, ao, be, bo = lax.fori_loop(
                0, HIST // 8, rbody, (zero, zero, zero, zero))
            obuf[r, pl.ds(0, LANES)] = (ae + ao) * inv
            obuf[r, pl.ds(LANES, LANES)] = (be + bo) * inv

        for p in range(NBUF):
            g_start(p, p)

        def body(i, carry):
            r0 = NBUF * i
            for p in range(NBUF):
                g_wait(r0 + p, p)
                reduce_row(r0 + p, p)
                g_start(r0 + NBUF + p, p)
            return carry

        lax.fori_loop(0, BPW // NBUF - 1, body, 0)

        r0 = BPW - NBUF
        for p in range(NBUF):
            g_wait(r0 + p, p)
            reduce_row(r0 + p, p)

        pltpu.sync_copy(obuf, out_hbm.at[pl.ds(w * BPW, BPW)])

    return k


_sc_kernel = _make_kernel()


def kernel(xs, table):
    xs_r = xs.astype(jnp.int32).reshape(NW, BPW * HIST)
    return _sc_kernel(xs_r, table)
